# Initial kernel scaffold; baseline (speedup 1.0000x reference)
import jax
import jax.numpy as jnp
from jax.experimental import pallas as pl


def kernel(x):
    raise NotImplementedError("write your pallas kernel here")


# speedup vs baseline: 9.4767x; 9.4767x over previous
import jax
import jax.numpy as jnp
from jax.experimental import pallas as pl
from jax.experimental.pallas import tpu as pltpu

_W = (
    (-4.0 / 15.0, -2.0 / 15.0, -1.0 / 15.0),
    (-1.0 / 30.0, 1.0, -1.0 / 30.0),
    (-1.0 / 15.0, -2.0 / 15.0, -4.0 / 15.0),
)
_EPS = 4e-5


def _plane_kernel(x_ref, o_ref):
    x = x_ref[0]
    H, W = x.shape
    zr = jnp.zeros((1, W), x.dtype)
    zc = jnp.zeros((H + 2, 1), x.dtype)
    xp = jnp.concatenate([zr, x, zr], axis=0)
    xp = jnp.concatenate([zc, xp, zc], axis=1)
    s = None
    for di in range(3):
        for dj in range(3):
            t = _W[di][dj] * xp[di:di + H, dj:dj + W]
            s = t if s is None else s + t
    m = jnp.mean(s)
    c = s - m
    v = jnp.mean(c * c)
    o_ref[0] = c * jax.lax.rsqrt(v + _EPS)


def kernel(x):
    B, C, H, W = x.shape
    xf = x.reshape(B * C, H, W)
    out = pl.pallas_call(
        _plane_kernel,
        grid=(B * C,),
        in_specs=[pl.BlockSpec((1, H, W), lambda i: (i, 0, 0))],
        out_specs=pl.BlockSpec((1, H, W), lambda i: (i, 0, 0)),
        out_shape=jax.ShapeDtypeStruct((B * C, H, W), x.dtype),
        compiler_params=pltpu.CompilerParams(
            dimension_semantics=("parallel",),
        ),
    )(xf)
    return out.reshape(B, C, H, W)


# speedup vs baseline: 21.8605x; 2.3068x over previous
import jax
import jax.numpy as jnp
from jax.experimental import pallas as pl
from jax.experimental.pallas import tpu as pltpu

_A1, _A2, _A3 = -4.0 / 15.0, -2.0 / 15.0, -1.0 / 15.0
_AM = -1.0 / 30.0
_EPS = 4e-5


def _plane_kernel(x_ref, o_ref):
    x = x_ref[...]
    N, H, W = x.shape
    col = jax.lax.broadcasted_iota(jnp.int32, (N, H, W), 2)
    xl = jnp.where(col == 0, 0.0, jnp.roll(x, 1, axis=2))
    xr = jnp.where(col == W - 1, 0.0, jnp.roll(x, -1, axis=2))
    ttop = _A1 * xl + _A2 * x + _A3 * xr
    tbot = _A3 * xl + _A2 * x + _A1 * xr
    tmid = x + _AM * (xl + xr)
    row = jax.lax.broadcasted_iota(jnp.int32, (N, H, W), 1)
    s = tmid
    s = s + jnp.where(row == 0, 0.0, jnp.roll(ttop, 1, axis=1))
    s = s + jnp.where(row == H - 1, 0.0, jnp.roll(tbot, -1, axis=1))
    m = jnp.mean(s, axis=(1, 2), keepdims=True)
    v = jnp.mean(s * s, axis=(1, 2), keepdims=True) - m * m
    o_ref[...] = (s - m) * jax.lax.rsqrt(v + _EPS)


def kernel(x):
    B, C, H, W = x.shape
    P = B * C
    N = 1
    xf = x.reshape(P, H, W)
    out = pl.pallas_call(
        _plane_kernel,
        grid=(P // N,),
        in_specs=[pl.BlockSpec((N, H, W), lambda i: (i, 0, 0))],
        out_specs=pl.BlockSpec((N, H, W), lambda i: (i, 0, 0)),
        out_shape=jax.ShapeDtypeStruct((P, H, W), x.dtype),
        compiler_params=pltpu.CompilerParams(
            dimension_semantics=("parallel",),
        ),
    )(xf)
    return out.reshape(B, C, H, W)


# speedup vs baseline: 25.9907x; 1.1889x over previous
import jax
import jax.numpy as jnp
from jax.experimental import pallas as pl
from jax.experimental.pallas import tpu as pltpu

_A1, _A2, _A3 = -4.0 / 15.0, -2.0 / 15.0, -1.0 / 15.0
_AM = -1.0 / 30.0
_EPS = 4e-5


def _plane_kernel(x_ref, o_ref):
    x = x_ref[...]
    N, H, W = x.shape
    col = jax.lax.broadcasted_iota(jnp.int32, (N, H, W), 2)
    xl = jnp.where(col == 0, 0.0, jnp.roll(x, 1, axis=2))
    xr = jnp.where(col == W - 1, 0.0, jnp.roll(x, -1, axis=2))
    ttop = _A1 * xl + _A2 * x + _A3 * xr
    tbot = _A3 * xl + _A2 * x + _A1 * xr
    tmid = x + _AM * (xl + xr)
    row = jax.lax.broadcasted_iota(jnp.int32, (N, H, W), 1)
    s = tmid
    s = s + jnp.where(row == 0, 0.0, jnp.roll(ttop, 1, axis=1))
    s = s + jnp.where(row == H - 1, 0.0, jnp.roll(tbot, -1, axis=1))
    m = jnp.mean(s, axis=(1, 2), keepdims=True)
    v = jnp.mean(s * s, axis=(1, 2), keepdims=True) - m * m
    o_ref[...] = (s - m) * jax.lax.rsqrt(v + _EPS)


def kernel(x):
    B, C, H, W = x.shape
    P = B * C
    N = 4
    xf = x.reshape(P, H, W)
    out = pl.pallas_call(
        _plane_kernel,
        grid=(P // N,),
        in_specs=[pl.BlockSpec((N, H, W), lambda i: (i, 0, 0))],
        out_specs=pl.BlockSpec((N, H, W), lambda i: (i, 0, 0)),
        out_shape=jax.ShapeDtypeStruct((P, H, W), x.dtype),
        compiler_params=pltpu.CompilerParams(
            dimension_semantics=("parallel",),
        ),
    )(xf)
    return out.reshape(B, C, H, W)
